# Initial kernel scaffold; baseline (speedup 1.0000x reference)
import jax
import jax.numpy as jnp
from jax.experimental import pallas as pl


def kernel(query_embed, passage_embed, top_k):
    raise NotImplementedError("write your pallas kernel here")


# speedup vs baseline: 1.2062x; 1.2062x over previous
import jax
import jax.numpy as jnp
from jax import lax
from jax.experimental import pallas as pl
from jax.experimental.pallas import tpu as pltpu
from jax.experimental.pallas import tpu_sc as plsc

NQ = 1024
D = 64
NP = 100000
KOUT = 10
G1 = 16
G2R = 16
G2 = G1 * G2R
NPP = 102400
NL1 = NPP // G1
NL2 = NPP // G2
NG = NP // G1
NEG = -3.0e38

QB = 128
PCH = 10240

NC, NS, NLANE = 2, 16, 16
NW = NC * NS
QPW = NQ // NW


def _tc_body(q_ref, p_ref, l1_ref):
    s = jnp.dot(q_ref[...], p_ref[...].T, preferred_element_type=jnp.float32)
    l1_ref[...] = jnp.max(s.reshape(QB, PCH // G1, G1), axis=-1)


_tc_call = pl.pallas_call(
    _tc_body,
    grid=(NQ // QB, NPP // PCH),
    in_specs=[
        pl.BlockSpec((QB, D), lambda i, j: (i, 0)),
        pl.BlockSpec((PCH, D), lambda i, j: (j, 0)),
    ],
    out_specs=[
        pl.BlockSpec((QB, PCH // G1), lambda i, j: (i, j)),
    ],
    out_shape=[
        jax.ShapeDtypeStruct((NQ, NL1), jnp.float32),
    ],
)


def _l2_body(l1_ref, l2_ref):
    col = lax.broadcasted_iota(jnp.int32, (QB, NL1), 1)
    l1m = jnp.where(col >= NG, NEG, l1_ref[...])
    l2_ref[...] = jnp.max(l1m.reshape(QB, NL2, G2R), axis=-1)


_l2_call = pl.pallas_call(
    _l2_body,
    grid=(NQ // QB,),
    in_specs=[pl.BlockSpec((QB, NL1), lambda i: (i, 0))],
    out_specs=[pl.BlockSpec((QB, NL2), lambda i: (i, 0))],
    out_shape=[jax.ShapeDtypeStruct((NQ, NL2), jnp.float32)],
)


def _merge16(tv, ti, cv, ci):
    cv, ci = plsc.sort_key_val(cv, ci, descending=True)
    rcv = lax.rev(cv, (0,))
    rci = lax.rev(ci, (0,))
    m = tv >= rcv
    mv = jnp.where(m, tv, rcv)
    mi = jnp.where(m, ti, rci)
    sv, si = plsc.sort_key_val(mv, mi, descending=True)
    return sv, si


def _bcast_lane(v, j):
    idx = jnp.full((NLANE,), j, jnp.int32)
    return v.at[idx].get(mode="promise_in_bounds")


def _sc_body(l1r_hbm, l2_hbm, p3_hbm, q_hbm, vals_hbm, inds_hbm,
             qrow_v, l2_v, l1g_v, blk_v, ov_v, oi_v, sem):
    cid = lax.axis_index("c")
    sid = lax.axis_index("s")
    wid = sid * NC + cid
    base = wid * QPW
    pltpu.sync_copy(q_hbm.at[pl.ds(base, QPW)], qrow_v)
    iota = lax.iota(jnp.int32, NLANE)
    negv = jnp.full((NLANE,), NEG, jnp.float32)
    zi = jnp.zeros((NLANE,), jnp.int32)

    def per_query(i, carry):
        q = base + i
        pltpu.sync_copy(l2_hbm.at[q], l2_v)
        ifull = jnp.full((NLANE,), i, jnp.int32)

        def l2_step(g, tvti):
            tv, ti = tvti
            cv = l2_v[pl.ds(g * NLANE, NLANE)]
            ci = g * NLANE + iota
            return _merge16(tv, ti, cv, ci)

        tv2, ti2 = lax.fori_loop(0, NL2 // NLANE, l2_step, (negv, zi))

        idx2 = q * NL2 + ti2
        pltpu.async_copy(l1r_hbm.at[idx2], l1g_v, sem).wait()

        def l1_step(j, tvti):
            tv, ti = tvti
            cv = l1g_v[j, :]
            ci = _bcast_lane(ti2, j) * G2R + iota
            cv = jnp.where(ci < NG, cv, NEG)
            return _merge16(tv, ti, cv, ci)

        tv1, ti1 = lax.fori_loop(0, NLANE, l1_step, (negv, zi))

        ti1c = jnp.clip(ti1, 0, NG - 1)
        pltpu.async_copy(p3_hbm.at[ti1c], blk_v, sem).wait()

        def rec_step(d, accs):
            dcol = jnp.full((NLANE,), d, jnp.int32)
            qbd = plsc.load_gather(qrow_v, [ifull, dcol])
            out = []
            for j in range(KOUT):
                jrow = jnp.full((NLANE,), j, jnp.int32)
                rows = plsc.load_gather(blk_v, [jrow, iota, dcol])
                out.append(accs[j] + rows * qbd)
            return tuple(out)

        accs = lax.fori_loop(
            0, D, rec_step, (jnp.zeros((NLANE,), jnp.float32),) * KOUT)

        fv, fi = negv, zi
        for j in range(KOUT):
            ci = _bcast_lane(ti1, j) * G1 + iota
            fv, fi = _merge16(fv, fi, accs[j], ci)

        ov_v[...] = fv
        oi_v[...] = fi
        pltpu.sync_copy(ov_v, vals_hbm.at[q])
        pltpu.sync_copy(oi_v, inds_hbm.at[q])
        return carry

    lax.fori_loop(0, QPW, per_query, 0)


_sc_mesh = plsc.VectorSubcoreMesh(
    core_axis_name="c", subcore_axis_name="s", num_cores=NC, num_subcores=NS)

_SC_OUT = [
    jax.ShapeDtypeStruct((NQ, NLANE), jnp.float32),
    jax.ShapeDtypeStruct((NQ, NLANE), jnp.int32),
]
_SC_SCRATCH = [
    pltpu.VMEM((QPW, D), jnp.float32),
    pltpu.VMEM((NL2,), jnp.float32),
    pltpu.VMEM((NLANE, G2R), jnp.float32),
    pltpu.VMEM((NLANE, G1, D), jnp.float32),
    pltpu.VMEM((NLANE,), jnp.float32),
    pltpu.VMEM((NLANE,), jnp.int32),
    pltpu.SemaphoreType.DMA,
]

_sc_call = pl.kernel(
    _sc_body,
    out_type=_SC_OUT,
    mesh=_sc_mesh,
    compiler_params=pltpu.CompilerParams(
        needs_layout_passes=False, use_tc_tiling_on_sc=False),
    scratch_types=_SC_SCRATCH,
)


def kernel(query_embed, passage_embed, top_k):
    qb16 = query_embed.astype(jnp.bfloat16)
    pb16 = passage_embed.astype(jnp.bfloat16)
    p_pad = jnp.pad(pb16, ((0, NPP - NP), (0, 0)))
    [l1] = _tc_call(qb16, p_pad)
    [l2] = _l2_call(l1)
    l1r = l1.reshape(NQ * NL2, G2R)
    p3 = pb16.astype(jnp.float32).reshape(NG, G1, D)
    vals16, inds16 = _sc_call(l1r, l2, p3, qb16.astype(jnp.float32))
    return inds16[:, :KOUT], vals16[:, :KOUT]


# speedup vs baseline: 3.9285x; 3.2571x over previous
import jax
import jax.numpy as jnp
from jax import lax
from jax.experimental import pallas as pl
from jax.experimental.pallas import tpu as pltpu
from jax.experimental.pallas import tpu_sc as plsc

NQ = 1024
D = 64
NP = 100000
KOUT = 10
NPP = 131072
NL1 = 8192
NL2 = 512
NEG = -3.0e38

QB = 128
PCH = 32768
NPC = NPP // PCH
L1W = PCH // 16
L2W = L1W // 16

NC, NS, NLANE = 2, 16, 16
NW = NC * NS
QPW = NQ // NW


def _tc_body(q_ref, p_ref, l1_ref, l2_ref):
    s = jnp.dot(q_ref[...], p_ref[...].T, preferred_element_type=jnp.float32)
    r = s.reshape(QB, 16, 16 * 128)
    l1 = jnp.max(r, axis=1)
    l1_ref[...] = l1
    l2_ref[...] = jnp.max(l1.reshape(QB, 16, 128), axis=1)


_tc_call = pl.pallas_call(
    _tc_body,
    grid=(NQ // QB, NPC),
    in_specs=[
        pl.BlockSpec((QB, D), lambda i, j: (i, 0)),
        pl.BlockSpec((PCH, D), lambda i, j: (j, 0)),
    ],
    out_specs=[
        pl.BlockSpec((QB, L1W), lambda i, j: (i, j)),
        pl.BlockSpec((QB, L2W), lambda i, j: (i, j)),
    ],
    out_shape=[
        jax.ShapeDtypeStruct((NQ, NL1), jnp.float32),
        jax.ShapeDtypeStruct((NQ, NL2), jnp.float32),
    ],
)


def _merge16(tv, ti, cv, ci):
    cv, ci = plsc.sort_key_val(cv, ci, descending=True)
    rcv = lax.rev(cv, (0,))
    rci = lax.rev(ci, (0,))
    m = tv >= rcv
    mv = jnp.where(m, tv, rcv)
    mi = jnp.where(m, ti, rci)
    sv, si = plsc.sort_key_val(mv, mi, descending=True)
    return sv, si


def _bcast_lane(v, j):
    idx = jnp.full((NLANE,), j, jnp.int32)
    return v.at[idx].get(mode="promise_in_bounds")


def _sc_body(l1_hbm, l2_hbm, p_hbm, q_hbm, vals_hbm, inds_hbm,
             qrow_v, l2a_v, l1row_v, blk_a, blk_b, idxu_v, idxa_v, idxb_v,
             ov_v, oi_v, sem, sem2):
    cid = lax.axis_index("c")
    sid = lax.axis_index("s")
    wid = sid * NC + cid
    base = wid * QPW
    pltpu.sync_copy(q_hbm.at[pl.ds(base, QPW)], qrow_v)
    pltpu.sync_copy(l2_hbm.at[pl.ds(base, QPW)], l2a_v)
    iota = lax.iota(jnp.int32, NLANE)
    negv = jnp.full((NLANE,), NEG, jnp.float32)
    zi = jnp.zeros((NLANE,), jnp.int32)

    def per_query(i, carry):
        q = base + i
        ifull = jnp.full((NLANE,), i, jnp.int32)
        cp_l1 = pltpu.make_async_copy(l1_hbm.at[q], l1row_v, sem2)
        cp_l1.start()

        def l2_step(g, tvti):
            tv, ti = tvti
            cv = l2a_v[i, pl.ds(g * NLANE, NLANE)]
            ci = g * NLANE + iota
            return _merge16(tv, ti, cv, ci)

        tv2, ti2 = lax.fori_loop(0, NL2 // NLANE, l2_step, (negv, zi))
        cp_l1.wait()

        tv1, ti1 = negv, zi
        for j in range(NLANE):
            g2 = _bcast_lane(ti2, j)
            g1m = (g2 >> 7) * L1W + (g2 & 127) + 128 * iota
            cv = plsc.load_gather(l1row_v, [g1m])
            tv1, ti1 = _merge16(tv1, ti1, cv, g1m)

        for j in range(NLANE):
            g1 = _bcast_lane(ti1, j)
            pid = (g1 >> 11) * PCH + (g1 & (L1W - 1)) + L1W * iota
            idxu_v[j, :] = pid
            pidc = jnp.minimum(pid, NP - 1)
            if j < 8:
                idxa_v[pl.ds(j * NLANE, NLANE)] = pidc
            else:
                idxb_v[pl.ds((j - 8) * NLANE, NLANE)] = pidc
        cp_a = pltpu.make_async_copy(p_hbm.at[idxa_v], blk_a, sem)
        cp_b = pltpu.make_async_copy(p_hbm.at[idxb_v], blk_b, sem)
        cp_a.start()
        cp_b.start()
        cp_a.wait()
        cp_b.wait()

        def rec_step(d, accs):
            dcol = jnp.full((NLANE,), d, jnp.int32)
            qbd = plsc.load_gather(qrow_v, [ifull, dcol])
            out = []
            for j in range(KOUT):
                blk = blk_a if j < 8 else blk_b
                row0 = (j if j < 8 else j - 8) * NLANE
                rows = plsc.load_gather(blk, [row0 + iota, dcol])
                out.append(accs[j] + rows * qbd)
            return tuple(out)

        accs = lax.fori_loop(
            0, D, rec_step, (jnp.zeros((NLANE,), jnp.float32),) * KOUT)

        fv, fi = negv, zi
        for j in range(KOUT):
            ci = idxu_v[j, :]
            sc = jnp.where(ci < NP, accs[j], NEG)
            fv, fi = _merge16(fv, fi, sc, ci)

        ov_v[...] = fv
        oi_v[...] = fi
        pltpu.sync_copy(ov_v, vals_hbm.at[q])
        pltpu.sync_copy(oi_v, inds_hbm.at[q])
        return carry

    lax.fori_loop(0, QPW, per_query, 0)


_sc_mesh = plsc.VectorSubcoreMesh(
    core_axis_name="c", subcore_axis_name="s", num_cores=NC, num_subcores=NS)

_SC_OUT = [
    jax.ShapeDtypeStruct((NQ, NLANE), jnp.float32),
    jax.ShapeDtypeStruct((NQ, NLANE), jnp.int32),
]
_SC_SCRATCH = [
    pltpu.VMEM((QPW, D), jnp.float32),
    pltpu.VMEM((QPW, NL2), jnp.float32),
    pltpu.VMEM((NL1,), jnp.float32),
    pltpu.VMEM((8 * NLANE, D), jnp.float32),
    pltpu.VMEM((8 * NLANE, D), jnp.float32),
    pltpu.VMEM((NLANE, NLANE), jnp.int32),
    pltpu.VMEM((8 * NLANE,), jnp.int32),
    pltpu.VMEM((8 * NLANE,), jnp.int32),
    pltpu.VMEM((NLANE,), jnp.float32),
    pltpu.VMEM((NLANE,), jnp.int32),
    pltpu.SemaphoreType.DMA,
    pltpu.SemaphoreType.DMA,
]

_sc_call = pl.kernel(
    _sc_body,
    out_type=_SC_OUT,
    mesh=_sc_mesh,
    compiler_params=pltpu.CompilerParams(
        needs_layout_passes=False, use_tc_tiling_on_sc=False),
    scratch_types=_SC_SCRATCH,
)


def kernel(query_embed, passage_embed, top_k):
    qb16 = query_embed.astype(jnp.bfloat16)
    pb16 = passage_embed.astype(jnp.bfloat16)
    p_pad = jnp.pad(pb16, ((0, NPP - NP), (0, 0)))
    l1, l2 = _tc_call(qb16, p_pad)
    p32 = pb16.astype(jnp.float32)
    vals16, inds16 = _sc_call(l1, l2, p32, qb16.astype(jnp.float32))
    return inds16[:, :KOUT], vals16[:, :KOUT]


# speedup vs baseline: 5.4729x; 1.3931x over previous
import jax
import jax.numpy as jnp
from jax import lax
from jax.experimental import pallas as pl
from jax.experimental.pallas import tpu as pltpu
from jax.experimental.pallas import tpu_sc as plsc

NQ = 1024
D = 64
NP = 100000
KOUT = 10
NPP = 131072
NL1 = 8192
NL2 = 512
NEG = -3.0e38

QB = 256
PCH = 32768
NPC = NPP // PCH
L1W = PCH // 16
L2W = L1W // 16

NC, NS, NLANE = 2, 16, 16
NW = NC * NS
QPW = NQ // NW


def _tc_body(q_ref, p_ref, l1_ref, l2_ref):
    s = jnp.dot(q_ref[...], p_ref[...].T, preferred_element_type=jnp.float32)
    l1 = s[:, :L1W]
    for a in range(1, 16):
        l1 = jnp.maximum(l1, s[:, a * L1W:(a + 1) * L1W])
    l1_ref[...] = l1
    l2 = l1[:, :L2W]
    for a in range(1, 16):
        l2 = jnp.maximum(l2, l1[:, a * L2W:(a + 1) * L2W])
    l2_ref[...] = l2


_tc_call = pl.pallas_call(
    _tc_body,
    grid=(NQ // QB, NPC),
    in_specs=[
        pl.BlockSpec((QB, D), lambda i, j: (i, 0)),
        pl.BlockSpec((PCH, D), lambda i, j: (j, 0)),
    ],
    out_specs=[
        pl.BlockSpec((QB, L1W), lambda i, j: (i, j)),
        pl.BlockSpec((QB, L2W), lambda i, j: (i, j)),
    ],
    out_shape=[
        jax.ShapeDtypeStruct((NQ, NL1), jnp.float32),
        jax.ShapeDtypeStruct((NQ, NL2), jnp.float32),
    ],
)


def _merge16(tv, ti, cv, ci):
    cv, ci = plsc.sort_key_val(cv, ci, descending=True)
    rcv = lax.rev(cv, (0,))
    rci = lax.rev(ci, (0,))
    m = tv >= rcv
    mv = jnp.where(m, tv, rcv)
    mi = jnp.where(m, ti, rci)
    sv, si = plsc.sort_key_val(mv, mi, descending=True)
    return sv, si


def _bcast_lane(v, j):
    idx = jnp.full((NLANE,), j, jnp.int32)
    return v.at[idx].get(mode="promise_in_bounds")


def _sc_body(l1_hbm, l2_hbm, p_hbm, q_hbm, vals_hbm, inds_hbm,
             qrow_v, l2a_v, l1row_v, blk_a, blk_b, idxu_v, idxa_v, idxb_v,
             ov_v, oi_v, sem, sem2):
    cid = lax.axis_index("c")
    sid = lax.axis_index("s")
    wid = sid * NC + cid
    base = wid * QPW
    pltpu.sync_copy(q_hbm.at[pl.ds(base, QPW)], qrow_v)
    pltpu.sync_copy(l2_hbm.at[pl.ds(base, QPW)], l2a_v)
    iota = lax.iota(jnp.int32, NLANE)
    negv = jnp.full((NLANE,), NEG, jnp.float32)
    zi = jnp.zeros((NLANE,), jnp.int32)

    def per_query(i, carry):
        q = base + i
        ifull = jnp.full((NLANE,), i, jnp.int32)
        cp_l1 = pltpu.make_async_copy(l1_hbm.at[q], l1row_v, sem2)
        cp_l1.start()

        def l2_step(g, tvti):
            tv, ti = tvti
            cv = l2a_v[i, pl.ds(g * NLANE, NLANE)]
            ci = g * NLANE + iota
            return _merge16(tv, ti, cv, ci)

        tv2, ti2 = lax.fori_loop(0, NL2 // NLANE, l2_step, (negv, zi))
        cp_l1.wait()

        tv1, ti1 = negv, zi
        for j in range(NLANE):
            g2 = _bcast_lane(ti2, j)
            g1m = (g2 >> 7) * L1W + (g2 & 127) + 128 * iota
            cv = plsc.load_gather(l1row_v, [g1m])
            tv1, ti1 = _merge16(tv1, ti1, cv, g1m)

        for j in range(NLANE):
            g1 = _bcast_lane(ti1, j)
            pid = (g1 >> 11) * PCH + (g1 & (L1W - 1)) + L1W * iota
            idxu_v[j, :] = pid
            pidc = jnp.minimum(pid, NP - 1)
            if j < 8:
                idxa_v[pl.ds(j * NLANE, NLANE)] = pidc
            else:
                idxb_v[pl.ds((j - 8) * NLANE, NLANE)] = pidc
        cp_a = pltpu.make_async_copy(p_hbm.at[idxa_v], blk_a, sem)
        cp_b = pltpu.make_async_copy(p_hbm.at[idxb_v], blk_b, sem)
        cp_a.start()
        cp_b.start()
        cp_a.wait()
        cp_b.wait()

        def rec_step(d, accs):
            dcol = jnp.full((NLANE,), d, jnp.int32)
            qbd = plsc.load_gather(qrow_v, [ifull, dcol])
            out = []
            for j in range(KOUT):
                blk = blk_a if j < 8 else blk_b
                row0 = (j if j < 8 else j - 8) * NLANE
                rows = plsc.load_gather(blk, [row0 + iota, dcol])
                out.append(accs[j] + rows * qbd)
            return tuple(out)

        accs = lax.fori_loop(
            0, D, rec_step, (jnp.zeros((NLANE,), jnp.float32),) * KOUT)

        fv, fi = negv, zi
        for j in range(KOUT):
            ci = idxu_v[j, :]
            sc = jnp.where(ci < NP, accs[j], NEG)
            fv, fi = _merge16(fv, fi, sc, ci)

        ov_v[...] = fv
        oi_v[...] = fi
        pltpu.sync_copy(ov_v, vals_hbm.at[q])
        pltpu.sync_copy(oi_v, inds_hbm.at[q])
        return carry

    lax.fori_loop(0, QPW, per_query, 0)


_sc_mesh = plsc.VectorSubcoreMesh(
    core_axis_name="c", subcore_axis_name="s", num_cores=NC, num_subcores=NS)

_SC_OUT = [
    jax.ShapeDtypeStruct((NQ, NLANE), jnp.float32),
    jax.ShapeDtypeStruct((NQ, NLANE), jnp.int32),
]
_SC_SCRATCH = [
    pltpu.VMEM((QPW, D), jnp.float32),
    pltpu.VMEM((QPW, NL2), jnp.float32),
    pltpu.VMEM((NL1,), jnp.float32),
    pltpu.VMEM((8 * NLANE, D), jnp.float32),
    pltpu.VMEM((8 * NLANE, D), jnp.float32),
    pltpu.VMEM((NLANE, NLANE), jnp.int32),
    pltpu.VMEM((8 * NLANE,), jnp.int32),
    pltpu.VMEM((8 * NLANE,), jnp.int32),
    pltpu.VMEM((NLANE,), jnp.float32),
    pltpu.VMEM((NLANE,), jnp.int32),
    pltpu.SemaphoreType.DMA,
    pltpu.SemaphoreType.DMA,
]

_sc_call = pl.kernel(
    _sc_body,
    out_type=_SC_OUT,
    mesh=_sc_mesh,
    compiler_params=pltpu.CompilerParams(
        needs_layout_passes=False, use_tc_tiling_on_sc=False),
    scratch_types=_SC_SCRATCH,
)


def kernel(query_embed, passage_embed, top_k):
    qb16 = query_embed.astype(jnp.bfloat16)
    pb16 = passage_embed.astype(jnp.bfloat16)
    p_pad = jnp.pad(pb16, ((0, NPP - NP), (0, 0)))
    l1, l2 = _tc_call(qb16, p_pad)
    p32 = pb16.astype(jnp.float32)
    vals16, inds16 = _sc_call(l1, l2, p32, qb16.astype(jnp.float32))
    return inds16[:, :KOUT], vals16[:, :KOUT]


# speedup vs baseline: 5.8218x; 1.0637x over previous
import jax
import jax.numpy as jnp
from jax import lax
from jax.experimental import pallas as pl
from jax.experimental.pallas import tpu as pltpu
from jax.experimental.pallas import tpu_sc as plsc

NQ = 1024
D = 64
NP = 100000
KOUT = 10
NPP = 131072
NL1 = 8192
NL2 = 512
NEG = -3.0e38

QB = 256
PCH = 32768
NPC = NPP // PCH
L1W = PCH // 16
L2W = L1W // 16

NC, NS, NLANE = 2, 16, 16
NW = NC * NS
QPW = NQ // NW


def _tc_body(q_ref, p_ref, l1_ref, l2_ref):
    s = jnp.dot(q_ref[...], p_ref[...].T, preferred_element_type=jnp.float32)
    l1 = s[:, :L1W]
    for a in range(1, 16):
        l1 = jnp.maximum(l1, s[:, a * L1W:(a + 1) * L1W])
    l1_ref[...] = l1
    l2 = l1[:, :L2W]
    for a in range(1, 16):
        l2 = jnp.maximum(l2, l1[:, a * L2W:(a + 1) * L2W])
    l2_ref[...] = l2


_tc_call = pl.pallas_call(
    _tc_body,
    grid=(NQ // QB, NPC),
    in_specs=[
        pl.BlockSpec((QB, D), lambda i, j: (i, 0)),
        pl.BlockSpec((PCH, D), lambda i, j: (j, 0)),
    ],
    out_specs=[
        pl.BlockSpec((QB, L1W), lambda i, j: (i, j)),
        pl.BlockSpec((QB, L2W), lambda i, j: (i, j)),
    ],
    out_shape=[
        jax.ShapeDtypeStruct((NQ, NL1), jnp.float32),
        jax.ShapeDtypeStruct((NQ, NL2), jnp.float32),
    ],
)


def _merge16(tv, ti, cv, ci):
    cv, ci = plsc.sort_key_val(cv, ci, descending=True)
    rcv = lax.rev(cv, (0,))
    rci = lax.rev(ci, (0,))
    m = tv >= rcv
    mv = jnp.where(m, tv, rcv)
    mi = jnp.where(m, ti, rci)
    sv, si = plsc.sort_key_val(mv, mi, descending=True)
    return sv, si


def _bcast_lane(v, j):
    idx = jnp.full((NLANE,), j, jnp.int32)
    return v.at[idx].get(mode="promise_in_bounds")


def _sc_body(l1_hbm, l2_hbm, p_hbm, q_hbm, vals_hbm, inds_hbm,
             qrow_v, l2a_v, l1row_v, qb_v, blk_a, blk_b, idxu_v, idxa_v,
             idxb_v, ov_v, oi_v, sem, sem2):
    cid = lax.axis_index("c")
    sid = lax.axis_index("s")
    wid = sid * NC + cid
    base = wid * QPW
    pltpu.sync_copy(q_hbm.at[pl.ds(base, QPW)], qrow_v)
    pltpu.sync_copy(l2_hbm.at[pl.ds(base, QPW)], l2a_v)
    iota = lax.iota(jnp.int32, NLANE)
    negv = jnp.full((NLANE,), NEG, jnp.float32)
    zi = jnp.zeros((NLANE,), jnp.int32)

    pltpu.make_async_copy(l1_hbm.at[base], l1row_v.at[0], sem2).start()

    def per_query(i, carry):
        q = base + i
        par = lax.rem(i, 2)
        ifull = jnp.full((NLANE,), i, jnp.int32)
        parf = jnp.full((NLANE,), par, jnp.int32)

        tv3, ti3 = negv, zi
        for k in range(8):
            cv = l2a_v[i, pl.ds(k * NLANE, NLANE)]
            for pc in range(1, NPC):
                cv = jnp.maximum(cv, l2a_v[i, pl.ds((pc * 8 + k) * NLANE, NLANE)])
            tv3, ti3 = _merge16(tv3, ti3, cv, k * NLANE + iota)

        tv2, ti2 = negv, zi
        for pc in range(NPC):
            ci = pc * L2W + ti3
            cv = plsc.load_gather(l2a_v, [ifull, ci])
            tv2, ti2 = _merge16(tv2, ti2, cv, ci)

        pltpu.make_async_copy(l1_hbm.at[q], l1row_v.at[par], sem2).wait()
        qn = jnp.minimum(q + 1, NQ - 1)
        pltpu.make_async_copy(l1_hbm.at[qn], l1row_v.at[1 - par], sem2).start()

        tv1, ti1 = negv, zi
        for j in range(NLANE):
            g2 = _bcast_lane(ti2, j)
            g1m = (g2 >> 7) * L1W + (g2 & 127) + 128 * iota
            cv = plsc.load_gather(l1row_v, [parf, g1m])
            tv1, ti1 = _merge16(tv1, ti1, cv, g1m)

        for j in range(NLANE):
            g1 = _bcast_lane(ti1, j)
            pid = (g1 >> 11) * PCH + (g1 & (L1W - 1)) + L1W * iota
            idxu_v[j, :] = pid
            pidc = jnp.minimum(pid, NP - 1)
            if j < 8:
                idxa_v[pl.ds(j * NLANE, NLANE)] = pidc
            else:
                idxb_v[pl.ds((j - 8) * NLANE, NLANE)] = pidc
        cp_a = pltpu.make_async_copy(p_hbm.at[idxa_v], blk_a, sem)
        cp_b = pltpu.make_async_copy(p_hbm.at[idxb_v], blk_b, sem)
        cp_a.start()
        cp_b.start()

        def qb_step(d, c):
            dcol = jnp.full((NLANE,), d, jnp.int32)
            qb_v[d, :] = plsc.load_gather(qrow_v, [ifull, dcol])
            return c

        lax.fori_loop(0, D, qb_step, 0)
        cp_a.wait()
        cp_b.wait()

        def rec_step(d, accs):
            dcol = jnp.full((NLANE,), d, jnp.int32)
            qbd = qb_v[d, :]
            out = []
            for j in range(KOUT):
                blk = blk_a if j < 8 else blk_b
                row0 = (j if j < 8 else j - 8) * NLANE
                rows = plsc.load_gather(blk, [row0 + iota, dcol])
                out.append(accs[j] + rows * qbd)
            return tuple(out)

        accs = lax.fori_loop(
            0, D, rec_step, (jnp.zeros((NLANE,), jnp.float32),) * KOUT)

        fv, fi = negv, zi
        for j in range(KOUT):
            ci = idxu_v[j, :]
            sc = jnp.where(ci < NP, accs[j], NEG)
            fv, fi = _merge16(fv, fi, sc, ci)

        ov_v[...] = fv
        oi_v[...] = fi
        pltpu.sync_copy(ov_v, vals_hbm.at[q])
        pltpu.sync_copy(oi_v, inds_hbm.at[q])
        return carry

    lax.fori_loop(0, QPW, per_query, 0)
    pltpu.make_async_copy(
        l1_hbm.at[jnp.minimum(base + QPW, NQ - 1)],
        l1row_v.at[QPW % 2], sem2).wait()


_sc_mesh = plsc.VectorSubcoreMesh(
    core_axis_name="c", subcore_axis_name="s", num_cores=NC, num_subcores=NS)

_SC_OUT = [
    jax.ShapeDtypeStruct((NQ, NLANE), jnp.float32),
    jax.ShapeDtypeStruct((NQ, NLANE), jnp.int32),
]
_SC_SCRATCH = [
    pltpu.VMEM((QPW, D), jnp.float32),
    pltpu.VMEM((QPW, NL2), jnp.float32),
    pltpu.VMEM((2, NL1), jnp.float32),
    pltpu.VMEM((D, NLANE), jnp.float32),
    pltpu.VMEM((8 * NLANE, D), jnp.float32),
    pltpu.VMEM((8 * NLANE, D), jnp.float32),
    pltpu.VMEM((NLANE, NLANE), jnp.int32),
    pltpu.VMEM((8 * NLANE,), jnp.int32),
    pltpu.VMEM((8 * NLANE,), jnp.int32),
    pltpu.VMEM((NLANE,), jnp.float32),
    pltpu.VMEM((NLANE,), jnp.int32),
    pltpu.SemaphoreType.DMA,
    pltpu.SemaphoreType.DMA,
]

_sc_call = pl.kernel(
    _sc_body,
    out_type=_SC_OUT,
    mesh=_sc_mesh,
    compiler_params=pltpu.CompilerParams(
        needs_layout_passes=False, use_tc_tiling_on_sc=False),
    scratch_types=_SC_SCRATCH,
)


def kernel(query_embed, passage_embed, top_k):
    qb16 = query_embed.astype(jnp.bfloat16)
    pb16 = passage_embed.astype(jnp.bfloat16)
    p_pad = jnp.pad(pb16, ((0, NPP - NP), (0, 0)))
    l1, l2 = _tc_call(qb16, p_pad)
    p32 = pb16.astype(jnp.float32)
    vals16, inds16 = _sc_call(l1, l2, p32, qb16.astype(jnp.float32))
    return inds16[:, :KOUT], vals16[:, :KOUT]


# speedup vs baseline: 5.8812x; 1.0102x over previous
import jax
import jax.numpy as jnp
from jax import lax
from jax.experimental import pallas as pl
from jax.experimental.pallas import tpu as pltpu
from jax.experimental.pallas import tpu_sc as plsc

NQ = 1024
D = 64
NP = 100000
KOUT = 10
NPP = 131072
NL1 = 8192
NL2 = 512
NEG = -3.0e38

QB = 256
PCH = 32768
NPC = NPP // PCH
L1W = PCH // 16
L2W = L1W // 16

NC, NS, NLANE = 2, 16, 16
NW = NC * NS
QPW = NQ // NW


def _tc_body(q_ref, p_ref, l1_ref, l2_ref):
    s = jnp.dot(q_ref[...], p_ref[...].T, preferred_element_type=jnp.float32)
    l1 = s[:, :L1W]
    for a in range(1, 16):
        l1 = jnp.maximum(l1, s[:, a * L1W:(a + 1) * L1W])
    l1_ref[...] = l1
    l2 = l1[:, :L2W]
    for a in range(1, 16):
        l2 = jnp.maximum(l2, l1[:, a * L2W:(a + 1) * L2W])
    l2_ref[...] = l2


_tc_call = pl.pallas_call(
    _tc_body,
    grid=(NQ // QB, NPC),
    in_specs=[
        pl.BlockSpec((QB, D), lambda i, j: (i, 0)),
        pl.BlockSpec((PCH, D), lambda i, j: (j, 0)),
    ],
    out_specs=[
        pl.BlockSpec((QB, L1W), lambda i, j: (i, j)),
        pl.BlockSpec((QB, L2W), lambda i, j: (i, j)),
    ],
    out_shape=[
        jax.ShapeDtypeStruct((NQ, NL1), jnp.float32),
        jax.ShapeDtypeStruct((NQ, NL2), jnp.float32),
    ],
)


def _merge16(tv, ti, cv, ci):
    cv, ci = plsc.sort_key_val(cv, ci, descending=True)
    rcv = lax.rev(cv, (0,))
    rci = lax.rev(ci, (0,))
    m = tv >= rcv
    mv = jnp.where(m, tv, rcv)
    mi = jnp.where(m, ti, rci)
    sv, si = plsc.sort_key_val(mv, mi, descending=True)
    return sv, si


def _bcast_lane(v, j):
    idx = jnp.full((NLANE,), j, jnp.int32)
    return v.at[idx].get(mode="promise_in_bounds")


def _sc_body(l1_hbm, l2_hbm, p_hbm, q_hbm, vals_hbm, inds_hbm,
             qrow_v, l2a_v, l1row_v, qb_v, blk_a1, blk_b1, blk_a2, blk_b2,
             idxu1_v, idxu2_v, idxa1_v, idxb1_v, idxa2_v, idxb2_v,
             ov_v, oi_v, sem_a, sem_b, sem_r1, sem_r2):
    cid = lax.axis_index("c")
    sid = lax.axis_index("s")
    wid = sid * NC + cid
    base = wid * QPW
    pltpu.sync_copy(q_hbm.at[pl.ds(base, QPW)], qrow_v)
    pltpu.sync_copy(l2_hbm.at[pl.ds(base, QPW)], l2a_v)
    iota = lax.iota(jnp.int32, NLANE)
    negv = jnp.full((NLANE,), NEG, jnp.float32)
    zi = jnp.zeros((NLANE,), jnp.int32)
    HP = QPW // 2

    def per_pair(i, carry):
        i2 = i + HP
        q1 = base + i
        q2 = base + i2
        if1 = jnp.full((NLANE,), i, jnp.int32)
        if2 = jnp.full((NLANE,), i2, jnp.int32)
        cp_r1 = pltpu.make_async_copy(l1_hbm.at[q1], l1row_v.at[0], sem_r1)
        cp_r2 = pltpu.make_async_copy(l1_hbm.at[q2], l1row_v.at[1], sem_r2)
        cp_r1.start()
        cp_r2.start()

        tva, tia = negv, zi
        tvb, tib = negv, zi
        for k in range(8):
            cva = l2a_v[i, pl.ds(k * NLANE, NLANE)]
            cvb = l2a_v[i2, pl.ds(k * NLANE, NLANE)]
            for pc in range(1, NPC):
                off = (pc * 8 + k) * NLANE
                cva = jnp.maximum(cva, l2a_v[i, pl.ds(off, NLANE)])
                cvb = jnp.maximum(cvb, l2a_v[i2, pl.ds(off, NLANE)])
            tva, tia = _merge16(tva, tia, cva, k * NLANE + iota)
            tvb, tib = _merge16(tvb, tib, cvb, k * NLANE + iota)

        tv2a, ti2a = negv, zi
        tv2b, ti2b = negv, zi
        for pc in range(NPC):
            cia = pc * L2W + tia
            cib = pc * L2W + tib
            cva = plsc.load_gather(l2a_v, [if1, cia])
            cvb = plsc.load_gather(l2a_v, [if2, cib])
            tv2a, ti2a = _merge16(tv2a, ti2a, cva, cia)
            tv2b, ti2b = _merge16(tv2b, ti2b, cvb, cib)

        cp_r1.wait()
        cp_r2.wait()
        z0 = jnp.zeros((NLANE,), jnp.int32)
        z1 = jnp.full((NLANE,), 1, jnp.int32)
        tv1a, ti1a = negv, zi
        tv1b, ti1b = negv, zi
        for j in range(NLANE):
            g2a = _bcast_lane(ti2a, j)
            g2b = _bcast_lane(ti2b, j)
            g1ma = (g2a >> 7) * L1W + (g2a & 127) + 128 * iota
            g1mb = (g2b >> 7) * L1W + (g2b & 127) + 128 * iota
            cva = plsc.load_gather(l1row_v, [z0, g1ma])
            cvb = plsc.load_gather(l1row_v, [z1, g1mb])
            tv1a, ti1a = _merge16(tv1a, ti1a, cva, g1ma)
            tv1b, ti1b = _merge16(tv1b, ti1b, cvb, g1mb)

        for j in range(NLANE):
            g1a = _bcast_lane(ti1a, j)
            g1b = _bcast_lane(ti1b, j)
            pida = (g1a >> 11) * PCH + (g1a & (L1W - 1)) + L1W * iota
            pidb = (g1b >> 11) * PCH + (g1b & (L1W - 1)) + L1W * iota
            idxu1_v[j, :] = pida
            idxu2_v[j, :] = pidb
            pca = jnp.minimum(pida, NP - 1)
            pcb = jnp.minimum(pidb, NP - 1)
            if j < 8:
                idxa1_v[pl.ds(j * NLANE, NLANE)] = pca
                idxa2_v[pl.ds(j * NLANE, NLANE)] = pcb
            else:
                idxb1_v[pl.ds((j - 8) * NLANE, NLANE)] = pca
                idxb2_v[pl.ds((j - 8) * NLANE, NLANE)] = pcb
        cps = [pltpu.make_async_copy(p_hbm.at[idxa1_v], blk_a1, sem_a),
               pltpu.make_async_copy(p_hbm.at[idxb1_v], blk_b1, sem_a),
               pltpu.make_async_copy(p_hbm.at[idxa2_v], blk_a2, sem_b),
               pltpu.make_async_copy(p_hbm.at[idxb2_v], blk_b2, sem_b)]
        for cp in cps:
            cp.start()

        def qb_step(d, c):
            d2 = 2 * d
            for dd in (d2, d2 + 1):
                dcol = jnp.full((NLANE,), dd, jnp.int32)
                qb_v[0, dd, :] = plsc.load_gather(qrow_v, [if1, dcol])
                qb_v[1, dd, :] = plsc.load_gather(qrow_v, [if2, dcol])
            return c

        lax.fori_loop(0, D // 2, qb_step, 0)
        for cp in cps:
            cp.wait()

        def rec_step(d, accs):
            accs1, accs2 = accs
            for dd in (2 * d, 2 * d + 1):
                dcol = jnp.full((NLANE,), dd, jnp.int32)
                qbd1 = qb_v[0, dd, :]
                qbd2 = qb_v[1, dd, :]
                o1, o2 = [], []
                for j in range(KOUT):
                    row0 = (j if j < 8 else j - 8) * NLANE + iota
                    r1 = plsc.load_gather(blk_a1 if j < 8 else blk_b1,
                                          [row0, dcol])
                    r2 = plsc.load_gather(blk_a2 if j < 8 else blk_b2,
                                          [row0, dcol])
                    o1.append(accs1[j] + r1 * qbd1)
                    o2.append(accs2[j] + r2 * qbd2)
                accs1, accs2 = tuple(o1), tuple(o2)
            return accs1, accs2

        z10 = (jnp.zeros((NLANE,), jnp.float32),) * KOUT
        accs1, accs2 = lax.fori_loop(0, D // 2, rec_step, (z10, z10))

        fva, fia = negv, zi
        fvb, fib = negv, zi
        for j in range(KOUT):
            cia = idxu1_v[j, :]
            cib = idxu2_v[j, :]
            sca = jnp.where(cia < NP, accs1[j], NEG)
            scb = jnp.where(cib < NP, accs2[j], NEG)
            fva, fia = _merge16(fva, fia, sca, cia)
            fvb, fib = _merge16(fvb, fib, scb, cib)

        ov_v[...] = fva
        oi_v[...] = fia
        pltpu.sync_copy(ov_v, vals_hbm.at[q1])
        pltpu.sync_copy(oi_v, inds_hbm.at[q1])
        ov_v[...] = fvb
        oi_v[...] = fib
        pltpu.sync_copy(ov_v, vals_hbm.at[q2])
        pltpu.sync_copy(oi_v, inds_hbm.at[q2])
        return carry

    lax.fori_loop(0, HP, per_pair, 0)


_sc_mesh = plsc.VectorSubcoreMesh(
    core_axis_name="c", subcore_axis_name="s", num_cores=NC, num_subcores=NS)

_SC_OUT = [
    jax.ShapeDtypeStruct((NQ, NLANE), jnp.float32),
    jax.ShapeDtypeStruct((NQ, NLANE), jnp.int32),
]
_SC_SCRATCH = [
    pltpu.VMEM((QPW, D), jnp.float32),
    pltpu.VMEM((QPW, NL2), jnp.float32),
    pltpu.VMEM((2, NL1), jnp.float32),
    pltpu.VMEM((2, D, NLANE), jnp.float32),
    pltpu.VMEM((8 * NLANE, D), jnp.float32),
    pltpu.VMEM((8 * NLANE, D), jnp.float32),
    pltpu.VMEM((8 * NLANE, D), jnp.float32),
    pltpu.VMEM((8 * NLANE, D), jnp.float32),
    pltpu.VMEM((NLANE, NLANE), jnp.int32),
    pltpu.VMEM((NLANE, NLANE), jnp.int32),
    pltpu.VMEM((8 * NLANE,), jnp.int32),
    pltpu.VMEM((8 * NLANE,), jnp.int32),
    pltpu.VMEM((8 * NLANE,), jnp.int32),
    pltpu.VMEM((8 * NLANE,), jnp.int32),
    pltpu.VMEM((NLANE,), jnp.float32),
    pltpu.VMEM((NLANE,), jnp.int32),
    pltpu.SemaphoreType.DMA,
    pltpu.SemaphoreType.DMA,
    pltpu.SemaphoreType.DMA,
    pltpu.SemaphoreType.DMA,
]

_sc_call = pl.kernel(
    _sc_body,
    out_type=_SC_OUT,
    mesh=_sc_mesh,
    compiler_params=pltpu.CompilerParams(
        needs_layout_passes=False, use_tc_tiling_on_sc=False),
    scratch_types=_SC_SCRATCH,
)


def kernel(query_embed, passage_embed, top_k):
    qb16 = query_embed.astype(jnp.bfloat16)
    pb16 = passage_embed.astype(jnp.bfloat16)
    p_pad = jnp.pad(pb16, ((0, NPP - NP), (0, 0)))
    l1, l2 = _tc_call(qb16, p_pad)
    p32 = pb16.astype(jnp.float32)
    vals16, inds16 = _sc_call(l1, l2, p32, qb16.astype(jnp.float32))
    return inds16[:, :KOUT], vals16[:, :KOUT]
